# Initial kernel scaffold; baseline (speedup 1.0000x reference)
import jax
import jax.numpy as jnp
from jax.experimental import pallas as pl


def kernel(h, x, edges, vel, edge_attr_orig, charges, emb, fW1, fb1, fW2, fb2, fW3, fb3, eW1, eb1, eW2, eb2, nW1, nb1, nW2, nb2):
    raise NotImplementedError("write your pallas kernel here")


# speedup vs baseline: 3.7068x; 3.7068x over previous
import functools

import jax
import jax.numpy as jnp
from jax import lax
from jax.experimental import pallas as pl
from jax.experimental.pallas import tpu as pltpu
from jax.experimental.pallas import tpu_sc as plsc

NC = 2
NS = 16
NW = NC * NS
IW = 125
CH = 8 * IW

F32 = jnp.float32


def _silu(v):
    return v * jax.nn.sigmoid(v)


def _node_prep_body(x_ref, vel_ref, ch_ref, emb_ref, fW1a_ref, fW1b_ref,
                    fb1_ref, fW2_ref, fb2_ref, fW3_ref, fb3_ref,
                    t_ref, pf_ref):
    x0 = x_ref[:, 0:1]
    x1 = x_ref[:, 1:2]
    v0 = vel_ref[:, 0:1]
    v1 = vel_ref[:, 1:2]
    n2 = v0 * v0 + v1 * v1
    inv = lax.rsqrt(jnp.maximum(n2, 1e-30))
    c = v0 * inv
    s = v1 * inv
    vl0 = c * v0 + s * v1
    vl1 = c * v1 - s * v0
    speed = jnp.sqrt(n2 + 1e-8)

    idx = jnp.clip(ch_ref[:, 0:1] + 1, 0, 2)
    e = emb_ref[...]
    emb_class = ((idx == 0).astype(F32) * e[0:1, :]
                 + (idx == 1).astype(F32) * e[1:2, :]
                 + (idx == 2).astype(F32) * e[2:3, :])
    w1a = fW1a_ref[...]
    pre = (fb1_ref[...]
           + x0 * w1a[0:1, :] + x1 * w1a[1:2, :]
           + v0 * w1a[2:3, :] + v1 * w1a[3:4, :]
           + jnp.dot(emb_class, fW1b_ref[...], preferred_element_type=F32))
    hdn = _silu(pre)
    hdn = _silu(jnp.dot(hdn, fW2_ref[...], preferred_element_type=F32)
                + fb2_ref[...])
    pf = jnp.dot(hdn, fW3_ref[...], preferred_element_type=F32) + fb3_ref[...]

    t_ref[...] = jnp.concatenate(
        [x0, x1, vl0, vl1, speed, c, s, jnp.zeros_like(x0)], axis=1)
    pf_ref[...] = pf


def _gather_body(rows_hbm, cols_hbm, t_hbm, gs_hbm, gd_hbm,
                 ir_v, ic_v, bs_v, bd_v, sem):
    wid = lax.axis_index("s") * NC + lax.axis_index("c")
    nch = rows_hbm.shape[0] // (NW * 8)

    def step(i, carry):
        r0 = wid * (nch * 8) + i * 8
        e0 = r0 * IW
        pltpu.sync_copy(rows_hbm.at[pl.ds(r0, 8)], ir_v)
        pltpu.sync_copy(cols_hbm.at[pl.ds(r0, 8)], ic_v)
        for j in range(8):
            pltpu.async_copy(t_hbm.at[ir_v.at[j]],
                             bs_v.at[pl.ds(j * IW, IW)], sem)
        for j in range(8):
            pltpu.async_copy(t_hbm.at[ic_v.at[j]],
                             bd_v.at[pl.ds(j * IW, IW)], sem)
        for j in range(8):
            pltpu.make_async_copy(t_hbm.at[ir_v.at[j]],
                                  bs_v.at[pl.ds(j * IW, IW)], sem).wait()
            pltpu.make_async_copy(t_hbm.at[ic_v.at[j]],
                                  bd_v.at[pl.ds(j * IW, IW)], sem).wait()
        pltpu.sync_copy(bs_v, gs_hbm.at[pl.ds(e0, CH)])
        pltpu.sync_copy(bd_v, gd_hbm.at[pl.ds(e0, CH)])
        return carry

    lax.fori_loop(0, nch, step, 0)


def _edge_body(gs_ref, gd_ref, ea_ref, eW1_ref, eb1_ref, eW2_ref, eb2_ref,
               m_ref):
    gs = gs_ref[...]
    gd = gd_ref[...]
    ea = ea_ref[...]
    dx0 = gs[:, 0:1] - gd[:, 0:1]
    dx1 = gs[:, 1:2] - gd[:, 1:2]
    c = gd[:, 5:6]
    s = gd[:, 6:7]
    rel0 = c * dx0 + s * dx1
    rel1 = c * dx1 - s * dx0
    dist = jnp.sqrt(rel0 * rel0 + rel1 * rel1 + 1e-8)
    cols = (gs[:, 2:3], gs[:, 3:4], gs[:, 4:5],
            gd[:, 2:3], gd[:, 3:4], gd[:, 4:5],
            rel0, rel1, dist,
            ea[:, 0:1], ea[:, 1:2], ea[:, 2:3], ea[:, 3:4])
    w1 = eW1_ref[...]
    pre = eb1_ref[...] + cols[0] * w1[0:1, :]
    for k in range(1, 13):
        pre = pre + cols[k] * w1[k:k + 1, :]
    hdn = _silu(pre)
    m_ref[...] = _silu(jnp.dot(hdn, eW2_ref[...], preferred_element_type=F32)
                       + eb2_ref[...])


def _scatter_body(cbase, colr_hbm, m_hbm, z_hbm, agg_hbm, ic_v, mb_v,
                  acc_sh, sem):
    cid = lax.axis_index("c")
    sid = lax.axis_index("s")
    nstripe = acc_sh.shape[0] // NS
    pltpu.sync_copy(z_hbm, acc_sh.at[pl.ds(sid * nstripe, nstripe)])
    plsc.subcore_barrier()

    rps = colr_hbm.shape[0] // NS
    nch = rps // 8

    def step(i, carry):
        r0 = sid * rps + i * 8
        e0 = r0 * IW
        pltpu.sync_copy(colr_hbm.at[pl.ds(r0, 8)], ic_v)
        pltpu.sync_copy(m_hbm.at[pl.ds(e0, CH),
                                 pl.ds(cbase + cid * 16, 16)], mb_v)
        for j in range(8):
            pltpu.sync_copy(mb_v.at[pl.ds(j * IW, IW)],
                            acc_sh.at[ic_v.at[j]], add=True)
        return carry

    lax.fori_loop(0, nch, step, 0)
    plsc.subcore_barrier()
    pltpu.sync_copy(acc_sh.at[pl.ds(sid * nstripe, nstripe)],
                    agg_hbm.at[pl.ds(sid * nstripe, nstripe),
                               pl.ds(cid * 16, 16)])


def _node_out_body(t_ref, pf_ref, agg0_ref, agg1_ref, nW1a_ref, nW1b0_ref,
                   nW1b1_ref, nb1_ref, nW2t_ref, nb2_ref, o_ref):
    t = t_ref[...]
    rf0 = t[:, 2:3]
    rf1 = t[:, 3:4]
    sp = t[:, 4:5]
    c = t[:, 5:6]
    s = t[:, 6:7]
    w1a = nW1a_ref[...]
    pre = (nb1_ref[...]
           + rf0 * w1a[0:1, :] + rf1 * w1a[1:2, :] + sp * w1a[2:3, :]
           + jnp.dot(agg0_ref[...], nW1b0_ref[...], preferred_element_type=F32)
           + jnp.dot(agg1_ref[...], nW1b1_ref[...],
                     preferred_element_type=F32))
    hdn = _silu(pre)
    w2t = nW2t_ref[...]
    p0 = jnp.sum(hdn * w2t[0:1, :], axis=1, keepdims=True) + nb2_ref[0:1, 0:1]
    p1 = jnp.sum(hdn * w2t[1:2, :], axis=1, keepdims=True) + nb2_ref[0:1, 1:2]
    pf = pf_ref[...]
    o0 = t[:, 0:1] + (c * p0 - s * p1) + pf[:, 0:1]
    o1 = t[:, 1:2] + (s * p0 + c * p1) + pf[:, 1:2]
    o_ref[...] = jnp.concatenate([o0, o1], axis=1)


def kernel(h, x, edges, vel, edge_attr_orig, charges, emb, fW1, fb1, fW2,
           fb2, fW3, fb3, eW1, eb1, eW2, eb2, nW1, nb1, nW2, nb2):
    del h
    N = x.shape[0]
    E = edges.shape[1]
    H = eW2.shape[1]
    BN = 2000 if N % 2000 == 0 else N
    BE = 2000 if E % 2000 == 0 else E

    full = lambda shp: pl.BlockSpec(shp, lambda i: tuple(0 for _ in shp))
    row2 = lambda w: pl.BlockSpec((BN, w), lambda i: (i, 0))

    T, pf = pl.pallas_call(
        _node_prep_body,
        grid=(N // BN,),
        in_specs=[row2(2), row2(2), row2(1), full((3, 16)), full((4, 32)),
                  full((16, 32)), full((32,)), full((32, 32)), full((32,)),
                  full((32, 2)), full((2,))],
        out_specs=[row2(8), row2(2)],
        out_shape=[jax.ShapeDtypeStruct((N, 8), F32),
                   jax.ShapeDtypeStruct((N, 2), F32)],
    )(x, vel, charges, emb, fW1[:4], fW1[4:], fb1, fW2, fb2, fW3, fb3)

    rows = edges[0].reshape(E // IW, IW)
    cols = edges[1].reshape(E // IW, IW)

    mesh = plsc.VectorSubcoreMesh(core_axis_name="c", subcore_axis_name="s",
                                  num_cores=NC, num_subcores=NS)
    sc_params = pltpu.CompilerParams(use_tc_tiling_on_sc=False)
    gather = pl.kernel(
        _gather_body,
        out_type=[jax.ShapeDtypeStruct((E, 8), F32),
                  jax.ShapeDtypeStruct((E, 8), F32)],
        mesh=mesh,
        scratch_types=[pltpu.VMEM((8, IW), jnp.int32),
                       pltpu.VMEM((8, IW), jnp.int32),
                       pltpu.VMEM((CH, 8), F32),
                       pltpu.VMEM((CH, 8), F32),
                       pltpu.SemaphoreType.DMA],
        compiler_params=sc_params,
    )
    gs, gd = gather(rows, cols, T)

    erow = lambda w: pl.BlockSpec((BE, w), lambda i: (i, 0))
    m = pl.pallas_call(
        _edge_body,
        grid=(E // BE,),
        in_specs=[erow(8), erow(8), erow(4), full((13, H)), full((H,)),
                  full((H, H)), full((H,))],
        out_specs=erow(H),
        out_shape=jax.ShapeDtypeStruct((E, H), F32),
    )(gs, gd, edge_attr_orig, eW1, eb1, eW2, eb2)

    zeros = jnp.zeros((N // NS, 16), F32)

    def _make_scatter(cbase):
        return pl.kernel(
            functools.partial(_scatter_body, cbase),
            out_type=jax.ShapeDtypeStruct((N, 32), F32),
            mesh=mesh,
            scratch_types=[pltpu.VMEM((8, IW), jnp.int32),
                           pltpu.VMEM((CH, 16), F32),
                           pltpu.VMEM_SHARED((N, 16), F32),
                           pltpu.SemaphoreType.DMA],
            compiler_params=sc_params,
        )

    agg0 = _make_scatter(0)(cols, m, zeros)
    agg1 = _make_scatter(32)(cols, m, zeros)

    out = pl.pallas_call(
        _node_out_body,
        grid=(N // BN,),
        in_specs=[row2(8), row2(2), row2(32), row2(32), full((3, H)),
                  full((32, H)), full((32, H)), full((H,)), full((2, H)),
                  full((1, 2))],
        out_specs=row2(2),
        out_shape=jax.ShapeDtypeStruct((N, 2), F32),
    )(T, pf, agg0, agg1, nW1[:3], nW1[3:35], nW1[35:67], nb1, nW2.T,
      nb2.reshape(1, 2))
    return out


# speedup vs baseline: 5.9591x; 1.6076x over previous
import functools

import jax
import jax.numpy as jnp
from jax import lax
from jax.experimental import pallas as pl
from jax.experimental.pallas import tpu as pltpu
from jax.experimental.pallas import tpu_sc as plsc

NC = 2
NS = 16
NW = NC * NS
IW = 125
CH = 8 * IW

F32 = jnp.float32


def _silu(v):
    return v * jax.nn.sigmoid(v)


def _node_prep_body(x_ref, vel_ref, ch_ref, emb_ref, fW1a_ref, fW1b_ref,
                    fb1_ref, fW2_ref, fb2_ref, fW3_ref, fb3_ref,
                    t_ref, pf_ref):
    x0 = x_ref[:, 0:1]
    x1 = x_ref[:, 1:2]
    v0 = vel_ref[:, 0:1]
    v1 = vel_ref[:, 1:2]
    n2 = v0 * v0 + v1 * v1
    inv = lax.rsqrt(jnp.maximum(n2, 1e-30))
    c = v0 * inv
    s = v1 * inv
    vl0 = c * v0 + s * v1
    vl1 = c * v1 - s * v0
    speed = jnp.sqrt(n2 + 1e-8)

    idx = jnp.clip(ch_ref[:, 0:1] + 1, 0, 2)
    e = emb_ref[...]
    emb_class = ((idx == 0).astype(F32) * e[0:1, :]
                 + (idx == 1).astype(F32) * e[1:2, :]
                 + (idx == 2).astype(F32) * e[2:3, :])
    w1a = fW1a_ref[...]
    pre = (fb1_ref[...]
           + x0 * w1a[0:1, :] + x1 * w1a[1:2, :]
           + v0 * w1a[2:3, :] + v1 * w1a[3:4, :]
           + jnp.dot(emb_class, fW1b_ref[...], preferred_element_type=F32))
    hdn = _silu(pre)
    hdn = _silu(jnp.dot(hdn, fW2_ref[...], preferred_element_type=F32)
                + fb2_ref[...])
    pf = jnp.dot(hdn, fW3_ref[...], preferred_element_type=F32) + fb3_ref[...]

    t_ref[...] = jnp.concatenate(
        [x0, x1, vl0, vl1, speed, c, s, jnp.zeros_like(x0)], axis=1)
    pf_ref[...] = pf


def _gather_body(rows_hbm, cols_hbm, t_hbm, gs_hbm, gd_hbm,
                 ir_v, ic_v, bs_v, bd_v, sem):
    wid = lax.axis_index("s") * NC + lax.axis_index("c")
    nch = rows_hbm.shape[0] // (NW * 8)

    def step(i, carry):
        r0 = wid * (nch * 8) + i * 8
        e0 = r0 * IW
        pltpu.sync_copy(rows_hbm.at[pl.ds(r0, 8)], ir_v)
        pltpu.sync_copy(cols_hbm.at[pl.ds(r0, 8)], ic_v)
        for j in range(8):
            pltpu.async_copy(t_hbm.at[ir_v.at[j]],
                             bs_v.at[pl.ds(j * IW, IW)], sem)
        for j in range(8):
            pltpu.async_copy(t_hbm.at[ic_v.at[j]],
                             bd_v.at[pl.ds(j * IW, IW)], sem)
        for j in range(8):
            pltpu.make_async_copy(t_hbm.at[ir_v.at[j]],
                                  bs_v.at[pl.ds(j * IW, IW)], sem).wait()
            pltpu.make_async_copy(t_hbm.at[ic_v.at[j]],
                                  bd_v.at[pl.ds(j * IW, IW)], sem).wait()
        pltpu.sync_copy(bs_v, gs_hbm.at[pl.ds(e0, CH)])
        pltpu.sync_copy(bd_v, gd_hbm.at[pl.ds(e0, CH)])
        return carry

    lax.fori_loop(0, nch, step, 0)


def _edge_body(gs_ref, gd_ref, ea_ref, eW1T_ref, eb1c_ref, eW2T_ref,
               eb2c_ref, m_ref):
    gsT = gs_ref[...].T
    gdT = gd_ref[...].T
    eaT = ea_ref[...].T
    dx0 = gsT[0:1, :] - gdT[0:1, :]
    dx1 = gsT[1:2, :] - gdT[1:2, :]
    c = gdT[5:6, :]
    s = gdT[6:7, :]
    rel0 = c * dx0 + s * dx1
    rel1 = c * dx1 - s * dx0
    dist = jnp.sqrt(rel0 * rel0 + rel1 * rel1 + 1e-8)
    m_inT = jnp.concatenate(
        [gsT[2:5, :], gdT[2:5, :], rel0, rel1, dist, eaT], axis=0)
    pre = jnp.dot(eW1T_ref[...], m_inT,
                  preferred_element_type=F32) + eb1c_ref[...]
    hdn = _silu(pre)
    h2 = _silu(jnp.dot(eW2T_ref[...], hdn, preferred_element_type=F32)
               + eb2c_ref[...])
    m_ref[...] = h2.T


def _scatter_body(cbase, colr_hbm, m_hbm, z_hbm, agg_hbm, ic_v, mb_v,
                  acc_sh, sem):
    cid = lax.axis_index("c")
    sid = lax.axis_index("s")
    nstripe = acc_sh.shape[0] // NS
    pltpu.sync_copy(z_hbm, acc_sh.at[pl.ds(sid * nstripe, nstripe)])
    plsc.subcore_barrier()

    rps = colr_hbm.shape[0] // NS
    nch = rps // 8

    def step(i, carry):
        r0 = sid * rps + i * 8
        e0 = r0 * IW
        pltpu.sync_copy(colr_hbm.at[pl.ds(r0, 8)], ic_v)
        pltpu.sync_copy(m_hbm.at[pl.ds(e0, CH),
                                 pl.ds(cbase + cid * 16, 16)], mb_v)
        for j in range(8):
            pltpu.sync_copy(mb_v.at[pl.ds(j * IW, IW)],
                            acc_sh.at[ic_v.at[j]], add=True)
        return carry

    lax.fori_loop(0, nch, step, 0)
    plsc.subcore_barrier()
    pltpu.sync_copy(acc_sh.at[pl.ds(sid * nstripe, nstripe)],
                    agg_hbm.at[pl.ds(sid * nstripe, nstripe),
                               pl.ds(cid * 16, 16)])


def _node_out_body(t_ref, pf_ref, agg0_ref, agg1_ref, nW1a_ref, nW1b0_ref,
                   nW1b1_ref, nb1_ref, nW2t_ref, nb2_ref, o_ref):
    t = t_ref[...]
    rf0 = t[:, 2:3]
    rf1 = t[:, 3:4]
    sp = t[:, 4:5]
    c = t[:, 5:6]
    s = t[:, 6:7]
    w1a = nW1a_ref[...]
    pre = (nb1_ref[...]
           + rf0 * w1a[0:1, :] + rf1 * w1a[1:2, :] + sp * w1a[2:3, :]
           + jnp.dot(agg0_ref[...], nW1b0_ref[...], preferred_element_type=F32)
           + jnp.dot(agg1_ref[...], nW1b1_ref[...],
                     preferred_element_type=F32))
    hdn = _silu(pre)
    w2t = nW2t_ref[...]
    p0 = jnp.sum(hdn * w2t[0:1, :], axis=1, keepdims=True) + nb2_ref[0:1, 0:1]
    p1 = jnp.sum(hdn * w2t[1:2, :], axis=1, keepdims=True) + nb2_ref[0:1, 1:2]
    pf = pf_ref[...]
    o0 = t[:, 0:1] + (c * p0 - s * p1) + pf[:, 0:1]
    o1 = t[:, 1:2] + (s * p0 + c * p1) + pf[:, 1:2]
    o_ref[...] = jnp.concatenate([o0, o1], axis=1)


def kernel(h, x, edges, vel, edge_attr_orig, charges, emb, fW1, fb1, fW2,
           fb2, fW3, fb3, eW1, eb1, eW2, eb2, nW1, nb1, nW2, nb2):
    del h
    N = x.shape[0]
    E = edges.shape[1]
    H = eW2.shape[1]
    BN = 2000 if N % 2000 == 0 else N
    BE = 2000 if E % 2000 == 0 else E

    full = lambda shp: pl.BlockSpec(shp, lambda i: tuple(0 for _ in shp))
    row2 = lambda w: pl.BlockSpec((BN, w), lambda i: (i, 0))

    T, pf = pl.pallas_call(
        _node_prep_body,
        grid=(N // BN,),
        in_specs=[row2(2), row2(2), row2(1), full((3, 16)), full((4, 32)),
                  full((16, 32)), full((32,)), full((32, 32)), full((32,)),
                  full((32, 2)), full((2,))],
        out_specs=[row2(8), row2(2)],
        out_shape=[jax.ShapeDtypeStruct((N, 8), F32),
                   jax.ShapeDtypeStruct((N, 2), F32)],
    )(x, vel, charges, emb, fW1[:4], fW1[4:], fb1, fW2, fb2, fW3, fb3)

    rows = edges[0].reshape(E // IW, IW)
    cols = edges[1].reshape(E // IW, IW)

    mesh = plsc.VectorSubcoreMesh(core_axis_name="c", subcore_axis_name="s",
                                  num_cores=NC, num_subcores=NS)
    sc_params = pltpu.CompilerParams(use_tc_tiling_on_sc=False)
    gather = pl.kernel(
        _gather_body,
        out_type=[jax.ShapeDtypeStruct((E, 8), F32),
                  jax.ShapeDtypeStruct((E, 8), F32)],
        mesh=mesh,
        scratch_types=[pltpu.VMEM((8, IW), jnp.int32),
                       pltpu.VMEM((8, IW), jnp.int32),
                       pltpu.VMEM((CH, 8), F32),
                       pltpu.VMEM((CH, 8), F32),
                       pltpu.SemaphoreType.DMA],
        compiler_params=sc_params,
    )
    gs, gd = gather(rows, cols, T)

    erow = lambda w: pl.BlockSpec((BE, w), lambda i: (i, 0))
    m = pl.pallas_call(
        _edge_body,
        grid=(E // BE,),
        in_specs=[erow(8), erow(8), erow(4), full((H, 13)), full((H, 1)),
                  full((H, H)), full((H, 1))],
        out_specs=erow(H),
        out_shape=jax.ShapeDtypeStruct((E, H), F32),
    )(gs, gd, edge_attr_orig, eW1.T, eb1.reshape(H, 1), eW2.T,
      eb2.reshape(H, 1))

    zeros = jnp.zeros((N // NS, 16), F32)

    def _make_scatter(cbase):
        return pl.kernel(
            functools.partial(_scatter_body, cbase),
            out_type=jax.ShapeDtypeStruct((N, 32), F32),
            mesh=mesh,
            scratch_types=[pltpu.VMEM((8, IW), jnp.int32),
                           pltpu.VMEM((CH, 16), F32),
                           pltpu.VMEM_SHARED((N, 16), F32),
                           pltpu.SemaphoreType.DMA],
            compiler_params=sc_params,
        )

    agg0 = _make_scatter(0)(cols, m, zeros)
    agg1 = _make_scatter(32)(cols, m, zeros)

    out = pl.pallas_call(
        _node_out_body,
        grid=(N // BN,),
        in_specs=[row2(8), row2(2), row2(32), row2(32), full((3, H)),
                  full((32, H)), full((32, H)), full((H,)), full((2, H)),
                  full((1, 2))],
        out_specs=row2(2),
        out_shape=jax.ShapeDtypeStruct((N, 2), F32),
    )(T, pf, agg0, agg1, nW1[:3], nW1[3:35], nW1[35:67], nb1, nW2.T,
      nb2.reshape(1, 2))
    return out
